# Initial kernel scaffold; baseline (speedup 1.0000x reference)
import functools

import jax
import jax.numpy as jnp
from jax import lax
from jax.experimental import pallas as pl
from jax.experimental.pallas import tpu as pltpu
from jax.experimental.pallas import tpu_sc as plsc

B = 320000
D = 128
NC = 2
NS = 16
NW = NC * NS
PER_W = B // NW
R = 80
NCH = PER_W // R
L = 16


def _body(table_hbm, conn_hbm, out_hbm, raw_v, idx_v, rows_v, sem):
    wid = lax.axis_index("s") * NC + lax.axis_index("c")
    base = wid * PER_W

    lane = lax.iota(jnp.int32, L)
    zeros = jnp.zeros((L,), jnp.int32)

    def chunk(c, carry):
        off = base + c * R
        pltpu.sync_copy(conn_hbm.at[pl.ds(off, R)], raw_v)
        for j in range(R // L):
            rows = lane + (j * L)
            vals = plsc.load_gather(raw_v, [rows, zeros])
            idx_v[pl.ds(j * L, L)] = vals
        pltpu.async_copy(table_hbm.at[idx_v], rows_v, sem).wait()
        pltpu.sync_copy(rows_v, out_hbm.at[pl.ds(off, R)])
        return carry

    lax.fori_loop(0, NCH, chunk, None)


def kernel(atom_matrix, connectivity):
    mesh = plsc.VectorSubcoreMesh(core_axis_name="c", subcore_axis_name="s")
    k = functools.partial(
        pl.kernel,
        mesh=mesh,
        out_type=jax.ShapeDtypeStruct((B, D), jnp.float32),
        scratch_types=[
            pltpu.VMEM((R, 2), jnp.int32),
            pltpu.VMEM((R,), jnp.int32),
            pltpu.VMEM((R, D), jnp.float32),
            pltpu.SemaphoreType.DMA,
        ],
    )(_body)
    return k(atom_matrix, connectivity)


# speedup vs baseline: 1.6730x; 1.6730x over previous
import functools

import jax
import jax.numpy as jnp
from jax import lax
from jax.experimental import pallas as pl
from jax.experimental.pallas import tpu as pltpu
from jax.experimental.pallas import tpu_sc as plsc

B = 320000
D = 128
NC = 2
NS = 16
NW = NC * NS
R = 64
NCHUNKS = B // R
L = 16


def _body(table_hbm, conn_hbm, out_hbm, raw_v, idx_v, rows_v, sem):
    wid = lax.axis_index("s") * NC + lax.axis_index("c")
    nfull = NCHUNKS // NW
    extra = NCHUNKS - nfull * NW
    n_mine = nfull + jnp.where(wid < extra, 1, 0)

    def chunk(k, carry):
        ci = wid + k * NW
        pltpu.sync_copy(conn_hbm.at[pl.ds(ci * (2 * R), 2 * R)], raw_v)
        for j in range(R // L):
            pos = (lax.iota(jnp.int32, L) + (j * L)) * 2
            idx_v[pl.ds(j * L, L)] = plsc.load_gather(raw_v, [pos])
        pltpu.async_copy(table_hbm.at[idx_v], rows_v, sem).wait()
        pltpu.sync_copy(rows_v, out_hbm.at[pl.ds(ci * R, R)])
        return carry

    lax.fori_loop(0, n_mine, chunk, None)


def kernel(atom_matrix, connectivity):
    mesh = plsc.VectorSubcoreMesh(core_axis_name="c", subcore_axis_name="s")
    k = functools.partial(
        pl.kernel,
        mesh=mesh,
        out_type=jax.ShapeDtypeStruct((B, D), jnp.float32),
        compiler_params=pltpu.CompilerParams(needs_layout_passes=False),
        scratch_types=[
            pltpu.VMEM((2 * R,), jnp.int32),
            pltpu.VMEM((R,), jnp.int32),
            pltpu.VMEM((R, D), jnp.float32),
            pltpu.SemaphoreType.DMA,
        ],
    )(_body)
    return k(atom_matrix, connectivity.reshape(-1))
